# baseline (speedup 1.0000x reference)
import jax
import jax.numpy as jnp
from jax import lax
from jax.experimental import pallas as pl
from jax.experimental.pallas import tpu as pltpu
from jax.experimental.pallas import tpu_sc as plsc

NC = 2
NS = 16
NW = NC * NS


def _wid():
    return lax.axis_index("s") * NC + lax.axis_index("c")


def _sc_emb_gather(rflat, cflat, embr_flat, embc_flat):
    NR = rflat.shape[0]
    E = cflat[0].shape[0]
    n_cols = len(cflat)
    D = embr_flat.shape[1]
    r_per_w = NR // NW
    c_per_w = E // NW
    K = 1024
    H = K // 2
    n_r = r_per_w // K
    n_c = c_per_w // K

    def body(ridx_hbm, c0_hbm, c1_hbm, c2_hbm, c3_hbm, embr_hbm, embc_hbm,
             xr_hbm, xc_hbm,
             ia0, ib0, ia1, ib1, ra0, rb0, ra1, rb1,
             ga0, gb0, ga1, gb1, sa0, sb0, sa1, sb1):
        w = _wid()
        idx_v = ((ia0, ib0), (ia1, ib1))
        rows_v = ((ra0, rb0), (ra1, rb1))
        gsem = ((ga0, gb0), (ga1, gb1))
        ssem = ((sa0, sb0), (sa1, sb1))
        col_hbm = (c0_hbm, c1_hbm, c2_hbm, c3_hbm)
        chunks = (
            [(ridx_hbm, w * r_per_w + k * K, embr_hbm, xr_hbm,
              w * r_per_w + k * K) for k in range(n_r)] +
            [(col_hbm[c], w * c_per_w + k * K, embc_hbm, xc_hbm,
              c * E + w * c_per_w + k * K)
             for c in range(n_cols) for k in range(n_c)])
        stores = [None, None]
        pending = None
        for t, (isrc, soff, tbl, dst, off) in enumerate(chunks):
            s = t % 2
            if stores[s] is not None:
                stores[s][0].wait()
                stores[s][1].wait()
                stores[s] = None
            pltpu.sync_copy(isrc.at[pl.ds(soff, H)], idx_v[s][0])
            pltpu.sync_copy(isrc.at[pl.ds(soff + H, H)], idx_v[s][1])
            g1 = pltpu.async_copy(tbl.at[idx_v[s][0]], rows_v[s][0], gsem[s][0])
            g2 = pltpu.async_copy(tbl.at[idx_v[s][1]], rows_v[s][1], gsem[s][1])
            if pending is not None:
                p1, p2, pdst, poff, ps = pending
                p1.wait()
                p2.wait()
                stores[ps] = (
                    pltpu.async_copy(rows_v[ps][0], pdst.at[pl.ds(poff, H)],
                                     ssem[ps][0]),
                    pltpu.async_copy(rows_v[ps][1],
                                     pdst.at[pl.ds(poff + H, H)],
                                     ssem[ps][1]),
                )
            pending = (g1, g2, dst, off, s)
        p1, p2, pdst, poff, ps = pending
        p1.wait()
        p2.wait()
        pltpu.sync_copy(rows_v[ps][0], pdst.at[pl.ds(poff, H)])
        pltpu.sync_copy(rows_v[ps][1], pdst.at[pl.ds(poff + H, H)])
        for st in stores:
            if st is not None:
                st[0].wait()
                st[1].wait()

    f = pl.kernel(
        body,
        out_type=(
            jax.ShapeDtypeStruct((NR, D), jnp.float32),
            jax.ShapeDtypeStruct((n_cols * E, D), jnp.float32),
        ),
        mesh=plsc.VectorSubcoreMesh(core_axis_name="c", subcore_axis_name="s"),
        compiler_params=pltpu.CompilerParams(use_tc_tiling_on_sc=False),
        scratch_types=[
            pltpu.VMEM((H,), jnp.int32),
            pltpu.VMEM((H,), jnp.int32),
            pltpu.VMEM((H,), jnp.int32),
            pltpu.VMEM((H,), jnp.int32),
            pltpu.VMEM((H, D), jnp.float32),
            pltpu.VMEM((H, D), jnp.float32),
            pltpu.VMEM((H, D), jnp.float32),
            pltpu.VMEM((H, D), jnp.float32),
            pltpu.SemaphoreType.DMA,
            pltpu.SemaphoreType.DMA,
            pltpu.SemaphoreType.DMA,
            pltpu.SemaphoreType.DMA,
            pltpu.SemaphoreType.DMA,
            pltpu.SemaphoreType.DMA,
            pltpu.SemaphoreType.DMA,
            pltpu.SemaphoreType.DMA,
        ],
    )
    return f(rflat, cflat[0], cflat[1], cflat[2], cflat[3],
             embr_flat, embc_flat)


def _tc_stats(cnum_flat, rn):
    E, NNC = cnum_flat.shape
    B, NNR = rn.shape
    BLK = 4096
    n_steps = E // BLK
    r_steps = B // BLK

    def body(cn_ref, rn_ref, sc_ref, sr_ref):
        i = pl.program_id(0)

        @pl.when(i == 0)
        def _():
            sc_ref[...] = jnp.zeros_like(sc_ref)
            sr_ref[...] = jnp.zeros_like(sr_ref)

        cn = cn_ref[...]
        sc_ref[0:1, :] += jnp.sum(cn, axis=0, keepdims=True)
        sc_ref[1:2, :] += jnp.sum(cn * cn, axis=0, keepdims=True)

        @pl.when(i < r_steps)
        def _():
            r = rn_ref[...]
            sr_ref[0:1, :] += jnp.sum(r, axis=0, keepdims=True)
            sr_ref[1:2, :] += jnp.sum(r * r, axis=0, keepdims=True)

    return pl.pallas_call(
        body,
        grid=(n_steps,),
        in_specs=[
            pl.BlockSpec((BLK, NNC), lambda i: (i, 0)),
            pl.BlockSpec((BLK, NNR), lambda i: (i % r_steps, 0)),
        ],
        out_specs=[
            pl.BlockSpec((8, NNC), lambda i: (0, 0)),
            pl.BlockSpec((8, NNR), lambda i: (0, 0)),
        ],
        out_shape=[
            jax.ShapeDtypeStruct((8, NNC), jnp.float32),
            jax.ShapeDtypeStruct((8, NNR), jnp.float32),
        ],
    )(cnum_flat, rn)


def _tc_main(xchild, cnum, rn, stats_c, stats_r, W_out, b_pad):
    NCC, B, F, D = xchild.shape
    XD = NCC * D
    NNC = cnum.shape[2]
    NNR = rn.shape[1]
    OUT = W_out.shape[0]
    BLK = 512
    n_steps = B // BLK
    n_child = float(F * B)
    eps = 1e-5

    def body(xc_ref, cn_ref, rn_ref, sc_ref, sr_ref, w_ref, b_ref,
             agg_ref, rnn_ref):
        mc = sc_ref[0:1, :] / n_child
        vc = sc_ref[1:2, :] / n_child - mc * mc
        inv_c = lax.rsqrt(vc + eps)
        w1 = w_ref[:, :XD]
        w2s = w_ref[:, XD:] * inv_c
        b2 = b_ref[0:1, :] - jax.lax.dot_general(
            mc * inv_c, w2s, (((1,), (1,)), ((), ())),
            preferred_element_type=jnp.float32)
        acc = jnp.zeros((BLK, OUT), jnp.float32)
        for j in range(F):
            nj = cn_ref[:, j, :]
            hj = jax.lax.dot_general(
                nj, w2s, (((1,), (1,)), ((), ())),
                preferred_element_type=jnp.float32)
            for k in range(NCC):
                xjk = xc_ref[k, :, j, :]
                hj += jax.lax.dot_general(
                    xjk, w1[:, k * D:(k + 1) * D], (((1,), (1,)), ((), ())),
                    preferred_element_type=jnp.float32)
            acc += jnp.maximum(hj + b2, 0.0)
        agg_ref[...] = acc * (1.0 / F)
        mr = sr_ref[0:1, :] / float(B)
        vr = sr_ref[1:2, :] / float(B) - mr * mr
        rnn_ref[...] = (rn_ref[...] - mr) * lax.rsqrt(vr + eps)

    return pl.pallas_call(
        body,
        grid=(n_steps,),
        in_specs=[
            pl.BlockSpec((NCC, BLK, F, D), lambda i: (0, i, 0, 0)),
            pl.BlockSpec((BLK, F, NNC), lambda i: (i, 0, 0)),
            pl.BlockSpec((BLK, NNR), lambda i: (i, 0)),
            pl.BlockSpec((8, NNC), lambda i: (0, 0)),
            pl.BlockSpec((8, NNR), lambda i: (0, 0)),
            pl.BlockSpec((OUT, XD + NNC), lambda i: (0, 0)),
            pl.BlockSpec((8, OUT), lambda i: (0, 0)),
        ],
        out_specs=[
            pl.BlockSpec((BLK, OUT), lambda i: (i, 0)),
            pl.BlockSpec((BLK, NNR), lambda i: (i, 0)),
        ],
        out_shape=[
            jax.ShapeDtypeStruct((B, OUT), jnp.float32),
            jax.ShapeDtypeStruct((B, NNR), jnp.float32),
        ],
    )(xchild, cnum, rn, stats_c, stats_r, W_out, b_pad)


def kernel(raw_idx, root_cat_feat, root_num_feat, child_map, child_cat_feat,
           child_num_feat, emb_root, emb_child, W_out, b_out):
    B = raw_idx.shape[0]
    F = child_map.shape[1]
    NCR, V, D = emb_root.shape
    NCC = emb_child.shape[0]
    NNC = child_num_feat.shape[1]

    idx = raw_idx.astype(jnp.int32)

    rcid = jnp.take(root_cat_feat.astype(jnp.int32), idx, axis=0)
    rn = jnp.take(root_num_feat, idx, axis=0)
    cm = jnp.take(child_map.astype(jnp.int32), idx, axis=0)
    expanded = cm.reshape(-1)
    ccid = jnp.take(child_cat_feat.astype(jnp.int32), expanded,
                    axis=0)
    cnum = jnp.take(child_num_feat, expanded, axis=0)

    rflat = (rcid + jnp.arange(NCR, dtype=jnp.int32) * V).reshape(-1)
    cols = tuple(ccid[:, k] + k * V for k in range(NCC))
    xroot, xchild = _sc_emb_gather(
        rflat, cols, emb_root.reshape(NCR * V, D),
        emb_child.reshape(NCC * V, D))

    stats_c, stats_r = _tc_stats(cnum, rn)
    b_pad = jnp.broadcast_to(b_out.reshape(1, -1), (8, b_out.shape[0]))
    E = B * F
    agg, rn_norm = _tc_main(xchild.reshape(NCC, B, F, D),
                            cnum.reshape(B, F, NNC),
                            rn, stats_c, stats_r, W_out, b_pad)
    return jnp.concatenate([xroot.reshape(B, NCR * D), rn_norm, agg], axis=1)


# speedup vs baseline: 1.1622x; 1.1622x over previous
import jax
import jax.numpy as jnp
from jax import lax
from jax.experimental import pallas as pl
from jax.experimental.pallas import tpu as pltpu
from jax.experimental.pallas import tpu_sc as plsc

NC = 2
NS = 16
NW = NC * NS


def _wid():
    return lax.axis_index("s") * NC + lax.axis_index("c")


def _sc_emb_gather(rflat, cflat, embr_flat, embc_flat):
    NR = rflat.shape[0]
    NCH = cflat.shape[0]
    D = embr_flat.shape[1]
    r_per_w = NR // NW
    c_per_w = NCH // NW
    K = 2048
    H = K // 2
    n_r = r_per_w // K
    n_c = c_per_w // K

    def body(ridx_hbm, cidx_hbm, embr_hbm, embc_hbm, xr_hbm, xc_hbm,
             ia0, ib0, ia1, ib1, ra0, rb0, ra1, rb1,
             ga0, gb0, ga1, gb1, sa0, sb0, sa1, sb1):
        w = _wid()
        idx_v = ((ia0, ib0), (ia1, ib1))
        rows_v = ((ra0, rb0), (ra1, rb1))
        gsem = ((ga0, gb0), (ga1, gb1))
        ssem = ((sa0, sb0), (sa1, sb1))
        chunks = (
            [(ridx_hbm, embr_hbm, xr_hbm, w * r_per_w + k * K)
             for k in range(n_r)] +
            [(cidx_hbm, embc_hbm, xc_hbm, w * c_per_w + k * K)
             for k in range(n_c)])
        stores = [None, None]
        pending = None
        for t, (isrc, tbl, dst, off) in enumerate(chunks):
            s = t % 2
            if stores[s] is not None:
                stores[s][0].wait()
                stores[s][1].wait()
                stores[s] = None
            pltpu.sync_copy(isrc.at[pl.ds(off, H)], idx_v[s][0])
            pltpu.sync_copy(isrc.at[pl.ds(off + H, H)], idx_v[s][1])
            g1 = pltpu.async_copy(tbl.at[idx_v[s][0]], rows_v[s][0], gsem[s][0])
            g2 = pltpu.async_copy(tbl.at[idx_v[s][1]], rows_v[s][1], gsem[s][1])
            if pending is not None:
                p1, p2, pdst, poff, ps = pending
                p1.wait()
                p2.wait()
                stores[ps] = (
                    pltpu.async_copy(rows_v[ps][0], pdst.at[pl.ds(poff, H)],
                                     ssem[ps][0]),
                    pltpu.async_copy(rows_v[ps][1],
                                     pdst.at[pl.ds(poff + H, H)],
                                     ssem[ps][1]),
                )
            pending = (g1, g2, dst, off, s)
        p1, p2, pdst, poff, ps = pending
        p1.wait()
        p2.wait()
        pltpu.sync_copy(rows_v[ps][0], pdst.at[pl.ds(poff, H)])
        pltpu.sync_copy(rows_v[ps][1], pdst.at[pl.ds(poff + H, H)])
        for st in stores:
            if st is not None:
                st[0].wait()
                st[1].wait()

    f = pl.kernel(
        body,
        out_type=(
            jax.ShapeDtypeStruct((NR, D), jnp.float32),
            jax.ShapeDtypeStruct((NCH, D), jnp.float32),
        ),
        mesh=plsc.VectorSubcoreMesh(core_axis_name="c", subcore_axis_name="s"),
        compiler_params=pltpu.CompilerParams(use_tc_tiling_on_sc=False),
        scratch_types=[
            pltpu.VMEM((H,), jnp.int32),
            pltpu.VMEM((H,), jnp.int32),
            pltpu.VMEM((H,), jnp.int32),
            pltpu.VMEM((H,), jnp.int32),
            pltpu.VMEM((H, D), jnp.float32),
            pltpu.VMEM((H, D), jnp.float32),
            pltpu.VMEM((H, D), jnp.float32),
            pltpu.VMEM((H, D), jnp.float32),
            pltpu.SemaphoreType.DMA,
            pltpu.SemaphoreType.DMA,
            pltpu.SemaphoreType.DMA,
            pltpu.SemaphoreType.DMA,
            pltpu.SemaphoreType.DMA,
            pltpu.SemaphoreType.DMA,
            pltpu.SemaphoreType.DMA,
            pltpu.SemaphoreType.DMA,
        ],
    )
    return f(rflat, cflat, embr_flat, embc_flat)


def _tc_stats(cnum_flat, rn):
    E, NNC = cnum_flat.shape
    B, NNR = rn.shape
    BLK = 4096
    n_steps = E // BLK
    r_steps = B // BLK

    def body(cn_ref, rn_ref, sc_ref, sr_ref):
        i = pl.program_id(0)

        @pl.when(i == 0)
        def _():
            sc_ref[...] = jnp.zeros_like(sc_ref)
            sr_ref[...] = jnp.zeros_like(sr_ref)

        cn = cn_ref[...]
        sc_ref[0:1, :] += jnp.sum(cn, axis=0, keepdims=True)
        sc_ref[1:2, :] += jnp.sum(cn * cn, axis=0, keepdims=True)

        @pl.when(i < r_steps)
        def _():
            r = rn_ref[...]
            sr_ref[0:1, :] += jnp.sum(r, axis=0, keepdims=True)
            sr_ref[1:2, :] += jnp.sum(r * r, axis=0, keepdims=True)

    return pl.pallas_call(
        body,
        grid=(n_steps,),
        in_specs=[
            pl.BlockSpec((BLK, NNC), lambda i: (i, 0)),
            pl.BlockSpec((BLK, NNR), lambda i: (i % r_steps, 0)),
        ],
        out_specs=[
            pl.BlockSpec((8, NNC), lambda i: (0, 0)),
            pl.BlockSpec((8, NNR), lambda i: (0, 0)),
        ],
        out_shape=[
            jax.ShapeDtypeStruct((8, NNC), jnp.float32),
            jax.ShapeDtypeStruct((8, NNR), jnp.float32),
        ],
    )(cnum_flat, rn)


def _tc_main(xchild, cnum, rn, stats_c, stats_r, W_out, b_pad):
    B, F, XD = xchild.shape
    NNC = cnum.shape[2]
    NNR = rn.shape[1]
    OUT = W_out.shape[0]
    BLK = 512
    n_steps = B // BLK
    n_child = float(F * B)
    eps = 1e-5

    def body(xc_ref, cn_ref, rn_ref, sc_ref, sr_ref, w_ref, b_ref,
             agg_ref, rnn_ref):
        mc = sc_ref[0:1, :] / n_child
        vc = sc_ref[1:2, :] / n_child - mc * mc
        inv_c = lax.rsqrt(vc + eps)
        w1 = w_ref[:, :XD]
        w2s = w_ref[:, XD:] * inv_c
        b2 = b_ref[0:1, :] - jax.lax.dot_general(
            mc * inv_c, w2s, (((1,), (1,)), ((), ())),
            preferred_element_type=jnp.float32)
        acc = jnp.zeros((BLK, OUT), jnp.float32)
        for j in range(F):
            xj = xc_ref[:, j, :]
            nj = cn_ref[:, j, :]
            hj = jax.lax.dot_general(
                xj, w1, (((1,), (1,)), ((), ())),
                preferred_element_type=jnp.float32)
            hj += jax.lax.dot_general(
                nj, w2s, (((1,), (1,)), ((), ())),
                preferred_element_type=jnp.float32)
            acc += jnp.maximum(hj + b2, 0.0)
        agg_ref[...] = acc * (1.0 / F)
        mr = sr_ref[0:1, :] / float(B)
        vr = sr_ref[1:2, :] / float(B) - mr * mr
        rnn_ref[...] = (rn_ref[...] - mr) * lax.rsqrt(vr + eps)

    return pl.pallas_call(
        body,
        grid=(n_steps,),
        in_specs=[
            pl.BlockSpec((BLK, F, XD), lambda i: (i, 0, 0)),
            pl.BlockSpec((BLK, F, NNC), lambda i: (i, 0, 0)),
            pl.BlockSpec((BLK, NNR), lambda i: (i, 0)),
            pl.BlockSpec((8, NNC), lambda i: (0, 0)),
            pl.BlockSpec((8, NNR), lambda i: (0, 0)),
            pl.BlockSpec((OUT, XD + NNC), lambda i: (0, 0)),
            pl.BlockSpec((8, OUT), lambda i: (0, 0)),
        ],
        out_specs=[
            pl.BlockSpec((BLK, OUT), lambda i: (i, 0)),
            pl.BlockSpec((BLK, NNR), lambda i: (i, 0)),
        ],
        out_shape=[
            jax.ShapeDtypeStruct((B, OUT), jnp.float32),
            jax.ShapeDtypeStruct((B, NNR), jnp.float32),
        ],
    )(xchild, cnum, rn, stats_c, stats_r, W_out, b_pad)


def kernel(raw_idx, root_cat_feat, root_num_feat, child_map, child_cat_feat,
           child_num_feat, emb_root, emb_child, W_out, b_out):
    B = raw_idx.shape[0]
    F = child_map.shape[1]
    NCR, V, D = emb_root.shape
    NCC = emb_child.shape[0]
    NNC = child_num_feat.shape[1]

    idx = raw_idx.astype(jnp.int32)

    rcid = jnp.take(root_cat_feat.astype(jnp.int32), idx, axis=0)
    rn = jnp.take(root_num_feat, idx, axis=0)
    cm = jnp.take(child_map.astype(jnp.int32), idx, axis=0)
    expanded = cm.reshape(-1)
    ccid = jnp.take(child_cat_feat.astype(jnp.int32), expanded,
                    axis=0)
    cnum = jnp.take(child_num_feat, expanded, axis=0)

    rflat = (rcid + jnp.arange(NCR, dtype=jnp.int32) * V).reshape(-1)
    cflat = (ccid + jnp.arange(NCC, dtype=jnp.int32) * V).reshape(-1)
    xroot, xchild = _sc_emb_gather(
        rflat, cflat, emb_root.reshape(NCR * V, D),
        emb_child.reshape(NCC * V, D))

    stats_c, stats_r = _tc_stats(cnum, rn)
    b_pad = jnp.broadcast_to(b_out.reshape(1, -1), (8, b_out.shape[0]))
    agg, rn_norm = _tc_main(xchild.reshape(B, F, NCC * D),
                            cnum.reshape(B, F, NNC),
                            rn, stats_c, stats_r, W_out, b_pad)
    return jnp.concatenate([xroot.reshape(B, NCR * D), rn_norm, agg], axis=1)
